# baseline (speedup 1.0000x reference)
import functools

import jax
import jax.numpy as jnp
from jax import lax
from jax.experimental import pallas as pl
from jax.experimental.pallas import tpu as pltpu
from jax.experimental.pallas import tpu_sc as plsc

_NC = 2
_NS = 16
_LANES = 16


def _mesh():
    return plsc.VectorSubcoreMesh(
        core_axis_name="c", subcore_axis_name="s", num_cores=_NC, num_subcores=_NS
    )


def _make_deg_kernel(e, npad):
    nw = _NC * _NS
    w = 2000
    iters = e // (nw * w)
    assert iters >= 5 and (iters - 2) % 3 == 0
    triples = (iters - 2) // 3
    seg = npad // _NS

    @functools.partial(
        pl.kernel,
        mesh=_mesh(),
        out_type=jax.ShapeDtypeStruct((_NC, npad), jnp.float32),
        scratch_types=[
            pltpu.VMEM((w,), jnp.int32),
            pltpu.VMEM((w,), jnp.int32),
            pltpu.VMEM((w,), jnp.int32),
            pltpu.VMEM((w,), jnp.float32),
            pltpu.VMEM_SHARED((npad,), jnp.float32),
            pltpu.SemaphoreType.DMA,
            pltpu.SemaphoreType.DMA,
            pltpu.SemaphoreType.DMA,
            pltpu.SemaphoreType.DMA,
            pltpu.SemaphoreType.DMA,
            pltpu.SemaphoreType.DMA,
        ],
    )
    def deg_kernel(e_hbm, zero_hbm, out_hbm, d0, d1, d2, onesv, accum,
                   l0, l1, l2, s0, s1, s2):
        cid = lax.axis_index("c")
        sid = lax.axis_index("s")
        wid = sid * _NC + cid
        row0 = wid * iters
        pltpu.sync_copy(zero_hbm.at[pl.ds(sid * seg, seg)],
                        accum.at[pl.ds(sid * seg, seg)])
        one16 = jnp.full((_LANES,), 1.0, jnp.float32)
        for k in range(w // _LANES):
            onesv[pl.ds(k * _LANES, _LANES)] = one16
        plsc.subcore_barrier()

        dbuf = (d0, d1, d2)
        lsem = (l0, l1, l2)
        ssem = (s0, s1, s2)

        def issue_load(g, s):
            pltpu.async_copy(e_hbm.at[1, row0 + g], dbuf[s], lsem[s])

        def wait_load(s):
            pltpu.make_async_copy(e_hbm.at[1, row0], dbuf[s], lsem[s]).wait()

        def issue_scat(s):
            pltpu.async_copy(onesv, accum.at[dbuf[s]], ssem[s], add=True)

        def wait_scat(s):
            pltpu.make_async_copy(onesv, accum.at[dbuf[s]], ssem[s]).wait()

        issue_load(0, 0)
        issue_load(1, 1)
        wait_load(0)
        issue_scat(0)
        issue_load(2, 2)
        wait_load(1)
        issue_scat(1)
        wait_scat(0)
        issue_load(3, 0)

        @pl.loop(0, triples)
        def _(t):
            g0 = 3 * t + 2
            for k, (cur, prev) in enumerate(((2, 1), (0, 2), (1, 0))):
                g = g0 + k
                wait_load(cur)
                issue_scat(cur)
                wait_scat(prev)
                gnext = jnp.minimum(g + 2, iters - 1)
                issue_load(gnext, prev)

        wait_scat(1)
        wait_load(2)
        wait_load(0)

        plsc.subcore_barrier()
        pltpu.sync_copy(accum.at[pl.ds(sid * seg, seg)],
                        out_hbm.at[cid, pl.ds(sid * seg, seg)])

    return deg_kernel


def _make_agg_kernel(e, npad):
    nw = _NC * _NS
    w = 2000
    iters = e // (nw * w)
    assert iters >= 5 and (iters - 2) % 3 == 0
    triples = (iters - 2) // 3
    seg = npad // _NS

    @functools.partial(
        pl.kernel,
        mesh=_mesh(),
        out_type=jax.ShapeDtypeStruct((_NC, npad), jnp.float32),
        scratch_types=[
            pltpu.VMEM((w,), jnp.int32),
            pltpu.VMEM((w,), jnp.int32),
            pltpu.VMEM((w,), jnp.int32),
            pltpu.VMEM((w,), jnp.int32),
            pltpu.VMEM((w,), jnp.int32),
            pltpu.VMEM((w,), jnp.int32),
            pltpu.VMEM((w,), jnp.float32),
            pltpu.VMEM((w,), jnp.float32),
            pltpu.VMEM((w,), jnp.float32),
            pltpu.VMEM((npad,), jnp.float32),
            pltpu.VMEM_SHARED((npad,), jnp.float32),
            pltpu.SemaphoreType.DMA,
            pltpu.SemaphoreType.DMA,
            pltpu.SemaphoreType.DMA,
            pltpu.SemaphoreType.DMA,
            pltpu.SemaphoreType.DMA,
            pltpu.SemaphoreType.DMA,
        ],
        compiler_params=pltpu.CompilerParams(needs_layout_passes=False),
    )
    def agg_kernel(e_hbm, tab_hbm, zero_hbm, out_hbm, sb0, sb1, sb2,
                   db0, db1, db2, vb0, vb1, vb2, table, accum,
                   l0, l1, l2, s0, s1, s2):
        cid = lax.axis_index("c")
        sid = lax.axis_index("s")
        wid = sid * _NC + cid
        row0 = wid * iters
        pltpu.sync_copy(zero_hbm.at[pl.ds(sid * seg, seg)],
                        accum.at[pl.ds(sid * seg, seg)])
        pltpu.sync_copy(tab_hbm, table)
        plsc.subcore_barrier()

        sbuf = (sb0, sb1, sb2)
        dbuf = (db0, db1, db2)
        vbuf = (vb0, vb1, vb2)
        lsem = (l0, l1, l2)
        ssem = (s0, s1, s2)

        def issue_load(g, s):
            pltpu.async_copy(e_hbm.at[0, row0 + g], sbuf[s], lsem[s])
            pltpu.async_copy(e_hbm.at[1, row0 + g], dbuf[s], lsem[s])

        def wait_load(s):
            pltpu.make_async_copy(e_hbm.at[0, row0], sbuf[s], lsem[s]).wait()
            pltpu.make_async_copy(e_hbm.at[1, row0], dbuf[s], lsem[s]).wait()

        def gather(s):
            for c in range(w // _LANES):
                idx = sbuf[s][pl.ds(c * _LANES, _LANES)]
                vals = plsc.load_gather(table, [idx])
                vbuf[s][pl.ds(c * _LANES, _LANES)] = vals

        def issue_scat(s):
            pltpu.async_copy(vbuf[s], accum.at[dbuf[s]], ssem[s], add=True)

        def wait_scat(s):
            pltpu.make_async_copy(vbuf[s], accum.at[dbuf[s]], ssem[s]).wait()

        issue_load(0, 0)
        issue_load(1, 1)
        wait_load(0)
        gather(0)
        issue_scat(0)
        issue_load(2, 2)
        wait_load(1)
        gather(1)
        issue_scat(1)
        wait_scat(0)
        issue_load(3, 0)

        @pl.loop(0, triples)
        def _(t):
            g0 = 3 * t + 2
            for k, (cur, prev) in enumerate(((2, 1), (0, 2), (1, 0))):
                g = g0 + k
                wait_load(cur)
                gather(cur)
                issue_scat(cur)
                wait_scat(prev)
                gnext = jnp.minimum(g + 2, iters - 1)
                issue_load(gnext, prev)

        wait_scat(1)
        wait_load(2)
        wait_load(0)

        plsc.subcore_barrier()
        pltpu.sync_copy(accum.at[pl.ds(sid * seg, seg)],
                        out_hbm.at[cid, pl.ds(sid * seg, seg)])

    return agg_kernel


def _tc_prep(deg_parts, xpad):
    r128 = xpad.shape

    def body(dref, xref, dis_ref, xd_ref):
        deg = dref[0] + dref[1]
        dis = jnp.where(deg > 0, lax.rsqrt(jnp.maximum(deg, 1e-12)),
                        jnp.zeros_like(deg))
        dis_ref[...] = dis
        xd_ref[...] = dis * xref[...]

    return pl.pallas_call(
        body,
        out_shape=(
            jax.ShapeDtypeStruct(r128, jnp.float32),
            jax.ShapeDtypeStruct(r128, jnp.float32),
        ),
    )(deg_parts, xpad)


def _tc_mid(t1_parts, dis, W1, b1, W2):
    r128 = dis.shape

    def body(tref, dis_ref, w1_ref, b1_ref, w2_ref, hd_ref):
        d = dis_ref[...]
        s1 = d * (tref[0] + tref[1])
        acc = jnp.zeros_like(s1)
        for j in range(16):
            acc = acc + jnp.maximum(s1 * w1_ref[0, j] + b1_ref[j], 0.0) * w2_ref[j, 0]
        hd_ref[...] = acc * d

    return pl.pallas_call(
        body,
        in_specs=[
            pl.BlockSpec(),
            pl.BlockSpec(),
            pl.BlockSpec(memory_space=pltpu.SMEM),
            pl.BlockSpec(memory_space=pltpu.SMEM),
            pl.BlockSpec(memory_space=pltpu.SMEM),
        ],
        out_shape=jax.ShapeDtypeStruct(r128, jnp.float32),
    )(t1_parts, dis, W1, b1, W2)


def _tc_final(t2_parts, dis, b2):
    r128 = dis.shape

    def body(tref, dis_ref, b2_ref, out_ref):
        out_ref[...] = dis_ref[...] * (tref[0] + tref[1]) + b2_ref[0]

    return pl.pallas_call(
        body,
        in_specs=[
            pl.BlockSpec(),
            pl.BlockSpec(),
            pl.BlockSpec(memory_space=pltpu.SMEM),
        ],
        out_shape=jax.ShapeDtypeStruct(r128, jnp.float32),
    )(t2_parts, dis, b2)


def kernel(x, edge_index, W1, b1, W2, b2):
    n = x.shape[0]
    e = edge_index.shape[1]
    npad = ((n + 1023) // 1024) * 1024
    r = npad // 128

    assert e % (_NC * _NS * 2000) == 0
    ei = edge_index.astype(jnp.int32).reshape(2, e // 2000, 2000)
    zeros_np = jnp.zeros((npad,), jnp.float32)
    xpad = jnp.concatenate([x[:, 0], jnp.zeros((npad - n,), jnp.float32)])

    deg_parts = _make_deg_kernel(e, npad)(ei, zeros_np)
    dis, xd = _tc_prep(deg_parts.reshape(2, r, 128), xpad.reshape(r, 128))

    agg = _make_agg_kernel(e, npad)
    t1_parts = agg(ei, xd.reshape(npad), zeros_np)
    hd = _tc_mid(t1_parts.reshape(2, r, 128), dis, W1, b1, W2)

    t2_parts = agg(ei, hd.reshape(npad), zeros_np)
    out = _tc_final(t2_parts.reshape(2, r, 128), dis, b2)

    return out.reshape(npad)[:n].reshape(n, 1)


# speedup vs baseline: 3.2145x; 3.2145x over previous
import functools

import jax
import jax.numpy as jnp
from jax import lax
from jax.experimental import pallas as pl
from jax.experimental.pallas import tpu as pltpu
from jax.experimental.pallas import tpu_sc as plsc

_NC = 2
_NS = 16
_LANES = 16


def _mesh():
    return plsc.VectorSubcoreMesh(
        core_axis_name="c", subcore_axis_name="s", num_cores=_NC, num_subcores=_NS
    )


def _make_deg_kernel(e, npad):
    nw = _NC * _NS
    w = 2000
    iters = e // (nw * w)
    assert iters >= 5 and (iters - 2) % 3 == 0
    triples = (iters - 2) // 3
    seg = npad // _NS

    @functools.partial(
        pl.kernel,
        mesh=_mesh(),
        out_type=jax.ShapeDtypeStruct((_NC, npad), jnp.float32),
        scratch_types=[
            pltpu.VMEM((w,), jnp.int32),
            pltpu.VMEM((w,), jnp.int32),
            pltpu.VMEM((w,), jnp.int32),
            pltpu.VMEM((w,), jnp.float32),
            pltpu.VMEM_SHARED((npad,), jnp.float32),
            pltpu.SemaphoreType.DMA,
            pltpu.SemaphoreType.DMA,
            pltpu.SemaphoreType.DMA,
            pltpu.SemaphoreType.DMA,
            pltpu.SemaphoreType.DMA,
            pltpu.SemaphoreType.DMA,
        ],
    )
    def deg_kernel(dst_hbm, zero_hbm, out_hbm, d0, d1, d2, onesv, accum,
                   l0, l1, l2, s0, s1, s2):
        cid = lax.axis_index("c")
        sid = lax.axis_index("s")
        wid = sid * _NC + cid
        base = wid * iters * w
        pltpu.sync_copy(zero_hbm.at[pl.ds(sid * seg, seg)],
                        accum.at[pl.ds(sid * seg, seg)])
        one16 = jnp.full((_LANES,), 1.0, jnp.float32)
        for k in range(w // _LANES):
            onesv[pl.ds(k * _LANES, _LANES)] = one16
        plsc.subcore_barrier()

        dbuf = (d0, d1, d2)
        lsem = (l0, l1, l2)
        ssem = (s0, s1, s2)

        def issue_load(g, s):
            pltpu.async_copy(dst_hbm.at[pl.ds(base + g * w, w)], dbuf[s],
                             lsem[s])

        def wait_load(s):
            pltpu.make_async_copy(dst_hbm.at[pl.ds(base, w)], dbuf[s],
                                  lsem[s]).wait()

        def issue_scat(s):
            pltpu.async_copy(onesv, accum.at[dbuf[s]], ssem[s], add=True)

        def wait_scat(s):
            pltpu.make_async_copy(onesv, accum.at[dbuf[s]], ssem[s]).wait()

        issue_load(0, 0)
        issue_load(1, 1)
        wait_load(0)
        issue_scat(0)
        issue_load(2, 2)
        wait_load(1)
        issue_scat(1)
        wait_scat(0)
        issue_load(3, 0)

        @pl.loop(0, triples)
        def _(t):
            g0 = 3 * t + 2
            for k, (cur, prev) in enumerate(((2, 1), (0, 2), (1, 0))):
                g = g0 + k
                wait_load(cur)
                issue_scat(cur)
                wait_scat(prev)
                gnext = jnp.minimum(g + 2, iters - 1)
                issue_load(gnext, prev)

        wait_scat(1)
        wait_load(2)
        wait_load(0)

        plsc.subcore_barrier()
        pltpu.sync_copy(accum.at[pl.ds(sid * seg, seg)],
                        out_hbm.at[cid, pl.ds(sid * seg, seg)])

    return deg_kernel


def _make_agg_kernel(e, npad):
    nw = _NC * _NS
    w = 2000
    iters = e // (nw * w)
    assert iters >= 5 and (iters - 2) % 3 == 0
    triples = (iters - 2) // 3
    seg = npad // _NS

    @functools.partial(
        pl.kernel,
        mesh=_mesh(),
        out_type=jax.ShapeDtypeStruct((_NC, npad), jnp.float32),
        scratch_types=[
            pltpu.VMEM((w,), jnp.int32),
            pltpu.VMEM((w,), jnp.int32),
            pltpu.VMEM((w,), jnp.int32),
            pltpu.VMEM((w,), jnp.int32),
            pltpu.VMEM((w,), jnp.int32),
            pltpu.VMEM((w,), jnp.int32),
            pltpu.VMEM((w,), jnp.float32),
            pltpu.VMEM((w,), jnp.float32),
            pltpu.VMEM((w,), jnp.float32),
            pltpu.VMEM((npad,), jnp.float32),
            pltpu.VMEM_SHARED((npad,), jnp.float32),
            pltpu.SemaphoreType.DMA,
            pltpu.SemaphoreType.DMA,
            pltpu.SemaphoreType.DMA,
            pltpu.SemaphoreType.DMA,
            pltpu.SemaphoreType.DMA,
            pltpu.SemaphoreType.DMA,
        ],
        compiler_params=pltpu.CompilerParams(needs_layout_passes=False),
    )
    def agg_kernel(src_hbm, dst_hbm, tab_hbm, zero_hbm, out_hbm, sb0, sb1, sb2,
                   db0, db1, db2, vb0, vb1, vb2, table, accum,
                   l0, l1, l2, s0, s1, s2):
        cid = lax.axis_index("c")
        sid = lax.axis_index("s")
        wid = sid * _NC + cid
        base = wid * iters * w
        pltpu.sync_copy(zero_hbm.at[pl.ds(sid * seg, seg)],
                        accum.at[pl.ds(sid * seg, seg)])
        pltpu.sync_copy(tab_hbm, table)
        plsc.subcore_barrier()

        sbuf = (sb0, sb1, sb2)
        dbuf = (db0, db1, db2)
        vbuf = (vb0, vb1, vb2)
        lsem = (l0, l1, l2)
        ssem = (s0, s1, s2)

        def issue_load(g, s):
            pltpu.async_copy(src_hbm.at[pl.ds(base + g * w, w)], sbuf[s],
                             lsem[s])
            pltpu.async_copy(dst_hbm.at[pl.ds(base + g * w, w)], dbuf[s],
                             lsem[s])

        def wait_load(s):
            pltpu.make_async_copy(src_hbm.at[pl.ds(base, w)], sbuf[s],
                                  lsem[s]).wait()
            pltpu.make_async_copy(dst_hbm.at[pl.ds(base, w)], dbuf[s],
                                  lsem[s]).wait()

        def gather(s):
            for c in range(w // _LANES):
                idx = sbuf[s][pl.ds(c * _LANES, _LANES)]
                vals = plsc.load_gather(table, [idx])
                vbuf[s][pl.ds(c * _LANES, _LANES)] = vals

        def issue_scat(s):
            pltpu.async_copy(vbuf[s], accum.at[dbuf[s]], ssem[s], add=True)

        def wait_scat(s):
            pltpu.make_async_copy(vbuf[s], accum.at[dbuf[s]], ssem[s]).wait()

        issue_load(0, 0)
        issue_load(1, 1)
        wait_load(0)
        gather(0)
        issue_scat(0)
        issue_load(2, 2)
        wait_load(1)
        gather(1)
        issue_scat(1)
        wait_scat(0)
        issue_load(3, 0)

        @pl.loop(0, triples)
        def _(t):
            g0 = 3 * t + 2
            for k, (cur, prev) in enumerate(((2, 1), (0, 2), (1, 0))):
                g = g0 + k
                wait_load(cur)
                gather(cur)
                issue_scat(cur)
                wait_scat(prev)
                gnext = jnp.minimum(g + 2, iters - 1)
                issue_load(gnext, prev)

        wait_scat(1)
        wait_load(2)
        wait_load(0)

        plsc.subcore_barrier()
        pltpu.sync_copy(accum.at[pl.ds(sid * seg, seg)],
                        out_hbm.at[cid, pl.ds(sid * seg, seg)])

    return agg_kernel


def _tc_prep(deg_parts, xpad):
    r128 = xpad.shape

    def body(dref, xref, dis_ref, xd_ref):
        deg = dref[0] + dref[1]
        dis = jnp.where(deg > 0, lax.rsqrt(jnp.maximum(deg, 1e-12)),
                        jnp.zeros_like(deg))
        dis_ref[...] = dis
        xd_ref[...] = dis * xref[...]

    return pl.pallas_call(
        body,
        out_shape=(
            jax.ShapeDtypeStruct(r128, jnp.float32),
            jax.ShapeDtypeStruct(r128, jnp.float32),
        ),
    )(deg_parts, xpad)


def _tc_mid(t1_parts, dis, W1, b1, W2):
    r128 = dis.shape

    def body(tref, dis_ref, w1_ref, b1_ref, w2_ref, hd_ref):
        d = dis_ref[...]
        s1 = d * (tref[0] + tref[1])
        acc = jnp.zeros_like(s1)
        for j in range(16):
            acc = acc + jnp.maximum(s1 * w1_ref[0, j] + b1_ref[j], 0.0) * w2_ref[j, 0]
        hd_ref[...] = acc * d

    return pl.pallas_call(
        body,
        in_specs=[
            pl.BlockSpec(),
            pl.BlockSpec(),
            pl.BlockSpec(memory_space=pltpu.SMEM),
            pl.BlockSpec(memory_space=pltpu.SMEM),
            pl.BlockSpec(memory_space=pltpu.SMEM),
        ],
        out_shape=jax.ShapeDtypeStruct(r128, jnp.float32),
    )(t1_parts, dis, W1, b1, W2)


def _tc_final(t2_parts, dis, b2):
    r128 = dis.shape

    def body(tref, dis_ref, b2_ref, out_ref):
        out_ref[...] = dis_ref[...] * (tref[0] + tref[1]) + b2_ref[0]

    return pl.pallas_call(
        body,
        in_specs=[
            pl.BlockSpec(),
            pl.BlockSpec(),
            pl.BlockSpec(memory_space=pltpu.SMEM),
        ],
        out_shape=jax.ShapeDtypeStruct(r128, jnp.float32),
    )(t2_parts, dis, b2)


def kernel(x, edge_index, W1, b1, W2, b2):
    n = x.shape[0]
    e = edge_index.shape[1]
    npad = ((n + 1023) // 1024) * 1024
    r = npad // 128

    assert e % (_NC * _NS * 2000) == 0
    ei = edge_index.astype(jnp.int32)
    srcf = ei[0]
    dstf = ei[1]
    zeros_np = jnp.zeros((npad,), jnp.float32)
    xpad = jnp.concatenate([x[:, 0], jnp.zeros((npad - n,), jnp.float32)])

    deg_parts = _make_deg_kernel(e, npad)(dstf, zeros_np)
    dis, xd = _tc_prep(deg_parts.reshape(2, r, 128), xpad.reshape(r, 128))

    agg = _make_agg_kernel(e, npad)
    t1_parts = agg(srcf, dstf, xd.reshape(npad), zeros_np)
    hd = _tc_mid(t1_parts.reshape(2, r, 128), dis, W1, b1, W2)

    t2_parts = agg(srcf, dstf, hd.reshape(npad), zeros_np)
    out = _tc_final(t2_parts.reshape(2, r, 128), dis, b2)

    return out.reshape(npad)[:n].reshape(n, 1)
